# Initial kernel scaffold; baseline (speedup 1.0000x reference)
import jax
import jax.numpy as jnp
from jax.experimental import pallas as pl


def kernel(adj_t, edges, emb, W1, a_src1, a_dst1, b1, bn_gamma, bn_beta, bn_mean, bn_var, W2, a_src2, a_dst2, b2, Wp1, bp1, Wp2, bp2):
    raise NotImplementedError("write your pallas kernel here")


# speedup vs baseline: 14.2855x; 14.2855x over previous
import functools

import jax
import jax.numpy as jnp
from jax import lax
from jax.experimental import pallas as pl
from jax.experimental.pallas import tpu as pltpu
from jax.experimental.pallas import tpu_sc as plsc

N = 10000
D = 128
DP = 144
NPAD = 10112
E = 320000
Q = 100000
CK = 128
NW = 32
ERows = (E + NW * CK - 1) // (NW * CK) * NW
REPT = ERows // NW
QRows = (Q + NW * CK - 1) // (NW * CK) * NW
RQPT = QRows // NW
EPAD = ERows * CK
QPAD = QRows * CK
ROWS_PER_TILE = NPAD // 16
NEG = 0.2

_f32 = jnp.float32
_i32 = jnp.int32


def _t1_body(x_ref, w_ref, a_ref, hp_ref, al_ref):
    _t1_body_from(x_ref[...], w_ref, a_ref, hp_ref, al_ref)


def _t2_body(acc_ref, den_ref, b_ref, gam_ref, bet_ref, mean_ref, var_ref,
             w_ref, a_ref, hp_ref, al_ref):
    accs = acc_ref[0] + acc_ref[1]
    dens = den_ref[0] + den_ref[1]
    x = accs[0:N, :] / (dens[0:N, 0:1] + 1e-16) + b_ref[...]
    x = (x - mean_ref[...]) * lax.rsqrt(var_ref[...] + 1e-5) * gam_ref[...] \
        + bet_ref[...]
    x = jnp.maximum(x, 0.0)
    _t1_body_from(x, w_ref, a_ref, hp_ref, al_ref)


def _t1_body_from(x, w_ref, a_ref, hp_ref, al_ref):
    h = jnp.dot(x, w_ref[...], preferred_element_type=_f32)
    al = jnp.dot(h, a_ref[...], preferred_element_type=_f32)
    hp_ref[...] = h
    al_ref[0:N, :] = al
    al_ref[N:NPAD, :] = jnp.zeros((NPAD - N, 2), _f32)


def _t3_body(acc_ref, den_ref, b_ref, x_ref):
    accs = acc_ref[0] + acc_ref[1]
    dens = den_ref[0] + den_ref[1]
    x_ref[...] = accs[0:N, :] / (dens[0:N, 0:1] + 1e-16) + b_ref[...]


BQ = QPAD // 10


def _t4_body(prod_ref, w1_ref, b1_ref, w2_ref, b2_ref, out_ref):
    t = jnp.dot(prod_ref[...], w1_ref[...], preferred_element_type=_f32)
    t = jnp.maximum(t + b1_ref[...], 0.0)
    o = jnp.dot(t, w2_ref[...], preferred_element_type=_f32) + b2_ref[...]
    out_ref[...] = jax.nn.sigmoid(o)


_MESH = plsc.VectorSubcoreMesh(core_axis_name="c", subcore_axis_name="s")
_SC_PARAMS = pltpu.CompilerParams(use_tc_tiling_on_sc=False,
                                  needs_layout_passes=False)


@functools.partial(
    pl.kernel,
    out_type=[jax.ShapeDtypeStruct((2, NPAD, D), _f32),
              jax.ShapeDtypeStruct((2, NPAD, 16), _f32)],
    mesh=_MESH,
    scratch_types=[
        pltpu.VMEM((2, CK), _i32),
        pltpu.VMEM((2 * NPAD,), _f32),
        pltpu.VMEM((CK, D), _f32),
        pltpu.VMEM((CK, 16), _f32),
        pltpu.VMEM_SHARED((NPAD, D), _f32),
        pltpu.VMEM_SHARED((NPAD, 16), _f32),
        pltpu.SemaphoreType.DMA,
    ],
    compiler_params=_SC_PARAMS,
)
def _sc_conv(hp_hbm, al_hbm, sd_hbm, acc_out, den_out,
             idx_v, al_v, rowg_v, wx_v, acc_sh, den_sh, sem):
    c = lax.axis_index("c")
    s = lax.axis_index("s")
    wid = c * 16 + s
    pltpu.sync_copy(al_hbm, al_v)

    zero16 = jnp.zeros((16,), _f32)

    def zrow(r, carry):
        rr = rowg_v.at[r]
        for q in range(D // 16):
            rr[pl.ds(q * 16, 16)] = zero16
        wx_v.at[r][pl.ds(0, 16)] = zero16
        return carry

    lax.fori_loop(0, CK, zrow, 0)

    base = s * ROWS_PER_TILE
    nfull = ROWS_PER_TILE // CK
    for k in range(nfull):
        pltpu.sync_copy(rowg_v, acc_sh.at[pl.ds(base + k * CK, CK)])
        pltpu.sync_copy(wx_v, den_sh.at[pl.ds(base + k * CK, CK)])
    rem = ROWS_PER_TILE - nfull * CK
    if rem:
        pltpu.sync_copy(rowg_v.at[pl.ds(0, rem)],
                        acc_sh.at[pl.ds(base + nfull * CK, rem)])
        pltpu.sync_copy(wx_v.at[pl.ds(0, rem)],
                        den_sh.at[pl.ds(base + nfull * CK, rem)])
    plsc.subcore_barrier()

    lane = lax.iota(_i32, 16)
    zeros_i = jnp.zeros((16,), _i32)

    def chunk(j, carry):
        pltpu.sync_copy(sd_hbm.at[wid, j], idx_v)
        cp = pltpu.async_copy(hp_hbm.at[idx_v.at[0]], rowg_v, sem)
        srow = idx_v.at[0]
        drow = idx_v.at[1]
        for g in range(CK // 16):
            s16 = srow[pl.ds(g * 16, 16)]
            d16 = drow[pl.ds(g * 16, 16)]
            als = plsc.load_gather(al_v, [s16 * 2])
            ald = plsc.load_gather(al_v, [d16 * 2 + 1])
            e = als + ald
            e = jnp.where(e > 0.0, e, NEG * e)
            plsc.store_scatter(wx_v, [lane + 16 * g, zeros_i], jnp.exp(e))
        cp.wait()

        def scale(r, cc):
            wr = plsc.load_gather(wx_v, [jnp.full((16,), r, _i32), zeros_i])
            rg = rowg_v.at[r]
            for q in range(D // 16):
                rg[pl.ds(q * 16, 16)] = rg[pl.ds(q * 16, 16)] * wr
            return cc

        lax.fori_loop(0, CK, scale, 0)
        pltpu.sync_copy(rowg_v, acc_sh.at[drow], add=True)
        pltpu.sync_copy(wx_v, den_sh.at[drow], add=True)
        return carry

    lax.fori_loop(0, REPT, chunk, 0)
    plsc.subcore_barrier()
    pltpu.sync_copy(acc_sh.at[pl.ds(base, ROWS_PER_TILE)],
                    acc_out.at[c, pl.ds(base, ROWS_PER_TILE)])
    pltpu.sync_copy(den_sh.at[pl.ds(base, ROWS_PER_TILE)],
                    den_out.at[c, pl.ds(base, ROWS_PER_TILE)])


@functools.partial(
    pl.kernel,
    out_type=jax.ShapeDtypeStruct((QPAD, D), _f32),
    mesh=_MESH,
    scratch_types=[
        pltpu.VMEM((RQPT, CK), _i32),
        pltpu.VMEM((RQPT, CK), _i32),
        pltpu.VMEM((CK, D), _f32),
        pltpu.VMEM((CK, D), _f32),
        pltpu.SemaphoreType.DMA,
        pltpu.SemaphoreType.DMA,
    ],
    compiler_params=_SC_PARAMS,
)
def _sc_link(x_hbm, ei_hbm, ej_hbm, prod_hbm, ei_v, ej_v, ra_v, rb_v,
             sema, semb):
    c = lax.axis_index("c")
    s = lax.axis_index("s")
    wid = c * 16 + s
    pltpu.sync_copy(ei_hbm.at[wid], ei_v)
    pltpu.sync_copy(ej_hbm.at[wid], ej_v)

    def chunk(j, carry):
        ca = pltpu.async_copy(x_hbm.at[ei_v.at[j]], ra_v, sema)
        cb = pltpu.async_copy(x_hbm.at[ej_v.at[j]], rb_v, semb)
        ca.wait()
        cb.wait()

        def mrow(r, cc):
            rra = ra_v.at[r]
            rrb = rb_v.at[r]
            for q in range(D // 16):
                rra[pl.ds(q * 16, 16)] = rra[pl.ds(q * 16, 16)] * \
                    rrb[pl.ds(q * 16, 16)]
            return cc

        lax.fori_loop(0, CK, mrow, 0)
        pltpu.sync_copy(ra_v, prod_hbm.at[pl.ds((wid * RQPT + j) * CK, CK)])
        return carry

    lax.fori_loop(0, RQPT, chunk, 0)


def kernel(adj_t, edges, emb, W1, a_src1, a_dst1, b1, bn_gamma, bn_beta,
           bn_mean, bn_var, W2, a_src2, a_dst2, b2, Wp1, bp1, Wp2, bp2):
    src = jnp.concatenate(
        [adj_t[0], jnp.zeros((EPAD - E,), _i32)]).reshape(NW, REPT, CK)
    dst = jnp.concatenate(
        [adj_t[1], jnp.full((EPAD - E,), N, _i32)]).reshape(NW, REPT, CK)
    sd = jnp.stack([src, dst], axis=2)
    ei = jnp.concatenate(
        [edges[0], jnp.zeros((QPAD - Q,), _i32)]).reshape(NW, RQPT, CK)
    ej = jnp.concatenate(
        [edges[1], jnp.zeros((QPAD - Q,), _i32)]).reshape(NW, RQPT, CK)
    a1 = jnp.stack([a_src1, a_dst1], axis=1)
    a2 = jnp.stack([a_src2, a_dst2], axis=1)
    b1r = b1.reshape(1, D)
    b2r = b2.reshape(1, D)
    gam = bn_gamma.reshape(1, D)
    bet = bn_beta.reshape(1, D)
    mean = bn_mean.reshape(1, D)
    var = bn_var.reshape(1, D)
    bp1r = bp1.reshape(1, D)
    bp2r = bp2.reshape(1, 1)

    hp1, al1 = pl.pallas_call(
        _t1_body,
        out_shape=[jax.ShapeDtypeStruct((N, D), _f32),
                   jax.ShapeDtypeStruct((NPAD, 2), _f32)],
    )(emb, W1, a1)

    acc1, den1 = _sc_conv(hp1, al1.reshape(2 * NPAD), sd)

    hp2, al2 = pl.pallas_call(
        _t2_body,
        out_shape=[jax.ShapeDtypeStruct((N, D), _f32),
                   jax.ShapeDtypeStruct((NPAD, 2), _f32)],
    )(acc1, den1, b1r, gam, bet, mean, var, W2, a2)

    acc2, den2 = _sc_conv(hp2, al2.reshape(2 * NPAD), sd)

    x2 = pl.pallas_call(
        _t3_body,
        out_shape=jax.ShapeDtypeStruct((N, D), _f32),
    )(acc2, den2, b2r)

    prod = _sc_link(x2, ei, ej)

    out = pl.pallas_call(
        _t4_body,
        grid=(QPAD // BQ,),
        in_specs=[
            pl.BlockSpec((BQ, D), lambda i: (i, 0)),
            pl.BlockSpec((D, D), lambda i: (0, 0)),
            pl.BlockSpec((1, D), lambda i: (0, 0)),
            pl.BlockSpec((D, 1), lambda i: (0, 0)),
            pl.BlockSpec((1, 1), lambda i: (0, 0)),
        ],
        out_specs=pl.BlockSpec((BQ, 1), lambda i: (i, 0)),
        out_shape=jax.ShapeDtypeStruct((QPAD, 1), _f32),
    )(prod, Wp1, bp1r, Wp2, bp2r)

    return out[:Q, 0]
